# Initial kernel scaffold; baseline (speedup 1.0000x reference)
import jax
import jax.numpy as jnp
from jax.experimental import pallas as pl


def kernel(x, W_qkv, W_proj, b_proj):
    raise NotImplementedError("write your pallas kernel here")


# speedup vs baseline: 7.2392x; 7.2392x over previous
import functools

import jax
import jax.numpy as jnp
from jax.experimental import pallas as pl

H = 12
M_LANDMARKS = 256
TOPK = 4
NEG = -1e30
BF = jnp.bfloat16


def _mm(a, b, dims):
    return jax.lax.dot_general(a.astype(BF), b.astype(BF), (dims, ((), ())),
                               preferred_element_type=jnp.float32)


def _qkv_kernel(x_ref, w_ref, o_ref):
    o_ref[...] = _mm(x_ref[...], w_ref[...], ((1,), (1,)))


def _proj_kernel(x_ref, w_ref, b_ref, o_ref):
    o_ref[...] = _mm(x_ref[...], w_ref[...], ((1,), (1,))) + b_ref[...]


def _attn_kernel(q_ref, k_ref, v_ref, o_ref, *, seg, scale):
    q = q_ref[0]
    k = k_ref[0]
    v = v_ref[0]
    bq = q.shape[0]
    n, hd = k.shape
    m = n // seg

    cent = jnp.mean(k.reshape(m, seg, hd), axis=1)

    rs = _mm(q, cent, ((1,), (1,))) * scale

    lane_m = jax.lax.broadcasted_iota(jnp.int32, (bq, m), 1)
    key_seg = jax.lax.broadcasted_iota(jnp.int32, (bq, n), 1) // seg
    masked = rs
    allow = jnp.zeros((bq, n), dtype=jnp.bool_)
    for _ in range(TOPK):
        mx = jnp.max(masked, axis=1, keepdims=True)
        eq = masked == mx
        idx = jnp.min(jnp.where(eq, lane_m, m), axis=1, keepdims=True)
        masked = jnp.where(lane_m == idx, NEG, masked)
        allow = allow | (key_seg == idx)

    s = _mm(q, k, ((1,), (1,))) * scale
    s = jnp.where(allow, s, NEG)
    mxs = jnp.max(s, axis=1, keepdims=True)
    e = jnp.exp(s - mxs)
    e = jnp.where(allow, e, 0.0)
    p = e / jnp.sum(e, axis=1, keepdims=True)
    o_ref[0] = _mm(p, v, ((1,), (0,)))


@functools.partial(jax.jit, static_argnames=("interpret",))
def kernel(x, W_qkv, W_proj, b_proj, interpret=False):
    Bb, Nn, Cc = x.shape
    hd = Cc // H
    scale = hd ** (-0.5)
    m = min(M_LANDMARKS, Nn)
    seg = (Nn + m - 1) // m

    xf = x.reshape(Bb * Nn, Cc)
    bn = Bb * Nn
    blk = 256
    grid_a = (bn // blk,)

    qkv = pl.pallas_call(
        _qkv_kernel,
        grid=grid_a,
        in_specs=[
            pl.BlockSpec((blk, Cc), lambda i: (i, 0)),
            pl.BlockSpec((3 * Cc, Cc), lambda i: (0, 0)),
        ],
        out_specs=pl.BlockSpec((blk, 3 * Cc), lambda i: (i, 0)),
        out_shape=jax.ShapeDtypeStruct((bn, 3 * Cc), jnp.float32),
        interpret=interpret,
    )(xf, W_qkv)

    qkv = qkv.reshape(Bb, Nn, 3, H, hd).transpose(2, 0, 3, 1, 4)
    qkv = qkv.reshape(3, Bb * H, Nn, hd)
    q, k, v = qkv[0], qkv[1], qkv[2]

    bq = 256
    grid_b = (Bb * H, Nn // bq)
    o = pl.pallas_call(
        functools.partial(_attn_kernel, seg=seg, scale=scale),
        grid=grid_b,
        in_specs=[
            pl.BlockSpec((1, bq, hd), lambda h, i: (h, i, 0)),
            pl.BlockSpec((1, Nn, hd), lambda h, i: (h, 0, 0)),
            pl.BlockSpec((1, Nn, hd), lambda h, i: (h, 0, 0)),
        ],
        out_specs=pl.BlockSpec((1, bq, hd), lambda h, i: (h, i, 0)),
        out_shape=jax.ShapeDtypeStruct((Bb * H, Nn, hd), jnp.float32),
        interpret=interpret,
    )(q, k, v)

    of = o.reshape(Bb, H, Nn, hd).transpose(0, 2, 1, 3).reshape(bn, Cc)

    out = pl.pallas_call(
        _proj_kernel,
        grid=grid_a,
        in_specs=[
            pl.BlockSpec((blk, Cc), lambda i: (i, 0)),
            pl.BlockSpec((Cc, Cc), lambda i: (0, 0)),
            pl.BlockSpec((1, Cc), lambda i: (0, 0)),
        ],
        out_specs=pl.BlockSpec((blk, Cc), lambda i: (i, 0)),
        out_shape=jax.ShapeDtypeStruct((bn, Cc), jnp.float32),
        interpret=interpret,
    )(of, W_proj, b_proj.reshape(1, Cc))

    return out.reshape(Bb, Nn, Cc)


# speedup vs baseline: 9.5857x; 1.3241x over previous
import functools

import jax
import jax.numpy as jnp
from jax.experimental import pallas as pl

H = 12
M_LANDMARKS = 256
TOPK = 4
NEG = -1e30
BF = jnp.bfloat16


def _mm(a, b, dims):
    return jax.lax.dot_general(a.astype(BF), b.astype(BF), (dims, ((), ())),
                               preferred_element_type=jnp.float32)


def _qkv_kernel(x_ref, w_ref, o_ref):
    o_ref[...] = _mm(x_ref[...], w_ref[...], ((1,), (1,)))


def _proj_kernel(x_ref, w_ref, b_ref, o_ref):
    o_ref[...] = _mm(x_ref[...], w_ref[...], ((1,), (1,))) + b_ref[...]


def _attn_kernel(q_ref, k_ref, v_ref, o_ref, *, seg, scale, hd):
    bq = q_ref.shape[0]
    n = k_ref.shape[0]
    m = n // seg
    lane_m = jax.lax.broadcasted_iota(jnp.int32, (bq, m), 1)
    key_seg = jax.lax.broadcasted_iota(jnp.int32, (bq, n), 1) // seg

    for half in range(2):
        sl = slice(half * hd, (half + 1) * hd)
        q = q_ref[:, sl]
        k = k_ref[:, sl]
        v = v_ref[:, sl]

        cent = jnp.mean(k.reshape(m, seg, hd), axis=1)

        rs = _mm(q, cent, ((1,), (1,))) * scale

        masked = rs
        allow = jnp.zeros((bq, n), dtype=jnp.bool_)
        for _ in range(TOPK):
            mx = jnp.max(masked, axis=1, keepdims=True)
            eq = masked == mx
            idx = jnp.min(jnp.where(eq, lane_m, m), axis=1, keepdims=True)
            masked = jnp.where(lane_m == idx, NEG, masked)
            allow = allow | (key_seg == idx)

        s = _mm(q, k, ((1,), (1,))) * scale
        s = jnp.where(allow, s, NEG)
        mxs = jnp.max(s, axis=1, keepdims=True)
        e = jnp.exp(s - mxs)
        e = jnp.where(allow, e, 0.0)
        p = e / jnp.sum(e, axis=1, keepdims=True)
        o_ref[:, sl] = _mm(p, v, ((1,), (0,)))


@functools.partial(jax.jit, static_argnames=("interpret",))
def kernel(x, W_qkv, W_proj, b_proj, interpret=False):
    Bb, Nn, Cc = x.shape
    hd = Cc // H
    scale = hd ** (-0.5)
    m = min(M_LANDMARKS, Nn)
    seg = (Nn + m - 1) // m

    xf = x.reshape(Bb * Nn, Cc)
    bn = Bb * Nn
    blk = 256
    grid_a = (bn // blk,)

    qkv = pl.pallas_call(
        _qkv_kernel,
        grid=grid_a,
        in_specs=[
            pl.BlockSpec((blk, Cc), lambda i: (i, 0)),
            pl.BlockSpec((3 * Cc, Cc), lambda i: (0, 0)),
        ],
        out_specs=pl.BlockSpec((blk, 3 * Cc), lambda i: (i, 0)),
        out_shape=jax.ShapeDtypeStruct((bn, 3 * Cc), jnp.float32),
        interpret=interpret,
    )(xf, W_qkv)

    bq = 256
    hp = H // 2
    grid_b = (hp, Nn // bq)
    of = pl.pallas_call(
        functools.partial(_attn_kernel, seg=seg, scale=scale, hd=hd),
        grid=grid_b,
        in_specs=[
            pl.BlockSpec((bq, 2 * hd), lambda h, i: (i, h)),
            pl.BlockSpec((Nn, 2 * hd), lambda h, i: (0, hp + h)),
            pl.BlockSpec((Nn, 2 * hd), lambda h, i: (0, 2 * hp + h)),
        ],
        out_specs=pl.BlockSpec((bq, 2 * hd), lambda h, i: (i, h)),
        out_shape=jax.ShapeDtypeStruct((bn, Cc), jnp.float32),
        interpret=interpret,
    )(qkv, qkv, qkv)

    out = pl.pallas_call(
        _proj_kernel,
        grid=grid_a,
        in_specs=[
            pl.BlockSpec((blk, Cc), lambda i: (i, 0)),
            pl.BlockSpec((Cc, Cc), lambda i: (0, 0)),
            pl.BlockSpec((1, Cc), lambda i: (0, 0)),
        ],
        out_specs=pl.BlockSpec((blk, Cc), lambda i: (i, 0)),
        out_shape=jax.ShapeDtypeStruct((bn, Cc), jnp.float32),
        interpret=interpret,
    )(of, W_proj, b_proj.reshape(1, Cc))

    return out.reshape(Bb, Nn, Cc)


# speedup vs baseline: 11.4351x; 1.1929x over previous
import functools

import jax
import jax.numpy as jnp
from jax.experimental import pallas as pl

H = 12
M_LANDMARKS = 256
TOPK = 4
NEG = -1e30
BIG = 1024.0
BF = jnp.bfloat16


def _mm(a, b, dims):
    return jax.lax.dot_general(a.astype(BF), b.astype(BF), (dims, ((), ())),
                               preferred_element_type=jnp.float32)


def _qkv_kernel(x_ref, w_ref, o_ref):
    o_ref[...] = _mm(x_ref[...], w_ref[...], ((1,), (1,)))


def _proj_kernel(x_ref, w_ref, b_ref, o_ref):
    o_ref[...] = _mm(x_ref[...], w_ref[...], ((1,), (1,))) + b_ref[...]


def _cent_kernel(k_ref, o_ref, *, seg):
    n, w = k_ref.shape
    o_ref[...] = jnp.mean(k_ref[...].reshape(n // seg, seg, w), axis=1)


def _attn_kernel(q_ref, k_ref, v_ref, c_ref, r_ref, o_ref, *, seg, scale, hd):
    bq = q_ref.shape[0]
    n = k_ref.shape[0]
    m = n // seg
    lane_m = jax.lax.broadcasted_iota(jnp.int32, (bq, m), 1)
    rbf = r_ref[...]

    for half in range(2):
        sl = slice(half * hd, (half + 1) * hd)
        q = q_ref[:, sl]
        k = k_ref[:, sl]
        v = v_ref[:, sl]
        cent = c_ref[:, sl]

        rs = _mm(q, cent, ((1,), (1,)))

        masked = rs
        selbig = jnp.zeros((bq, m), dtype=jnp.float32)
        for _ in range(TOPK):
            mx = jnp.max(masked, axis=1, keepdims=True)
            eq = masked == mx
            idx = jnp.min(jnp.where(eq, lane_m, m), axis=1, keepdims=True)
            hit = lane_m == idx
            masked = jnp.where(hit, NEG, masked)
            selbig = jnp.where(hit, BIG, selbig)

        s = _mm(q * scale, k, ((1,), (1,)))
        s = s + _mm(selbig, rbf, ((1,), (1,)))
        mxs = jnp.max(s, axis=1, keepdims=True)
        e = jnp.exp(s - mxs)
        p = e * (1.0 / jnp.sum(e, axis=1, keepdims=True))
        o_ref[:, sl] = _mm(p, v, ((1,), (0,)))


@functools.partial(jax.jit, static_argnames=("interpret",))
def kernel(x, W_qkv, W_proj, b_proj, interpret=False):
    Bb, Nn, Cc = x.shape
    hd = Cc // H
    scale = hd ** (-0.5)
    m = min(M_LANDMARKS, Nn)
    seg = (Nn + m - 1) // m

    xf = x.reshape(Bb * Nn, Cc)
    bn = Bb * Nn
    blk = 256
    grid_a = (bn // blk,)

    qkv = pl.pallas_call(
        _qkv_kernel,
        grid=grid_a,
        in_specs=[
            pl.BlockSpec((blk, Cc), lambda i: (i, 0)),
            pl.BlockSpec((3 * Cc, Cc), lambda i: (0, 0)),
        ],
        out_specs=pl.BlockSpec((blk, 3 * Cc), lambda i: (i, 0)),
        out_shape=jax.ShapeDtypeStruct((bn, 3 * Cc), jnp.float32),
        interpret=interpret,
    )(xf, W_qkv)

    hp = H // 2

    cent = pl.pallas_call(
        functools.partial(_cent_kernel, seg=seg),
        grid=(hp,),
        in_specs=[pl.BlockSpec((Nn, 2 * hd), lambda h: (0, hp + h))],
        out_specs=pl.BlockSpec((m, 2 * hd), lambda h: (0, h)),
        out_shape=jax.ShapeDtypeStruct((m, Cc), jnp.float32),
        interpret=interpret,
    )(qkv)

    rbf = (jnp.arange(Nn, dtype=jnp.int32)[:, None] // seg
           == jnp.arange(m, dtype=jnp.int32)[None, :]).astype(BF)

    bq = 256
    grid_b = (hp, Nn // bq)
    of = pl.pallas_call(
        functools.partial(_attn_kernel, seg=seg, scale=scale, hd=hd),
        grid=grid_b,
        in_specs=[
            pl.BlockSpec((bq, 2 * hd), lambda h, i: (i, h)),
            pl.BlockSpec((Nn, 2 * hd), lambda h, i: (0, hp + h)),
            pl.BlockSpec((Nn, 2 * hd), lambda h, i: (0, 2 * hp + h)),
            pl.BlockSpec((m, 2 * hd), lambda h, i: (0, h)),
            pl.BlockSpec((Nn, m), lambda h, i: (0, 0)),
        ],
        out_specs=pl.BlockSpec((bq, 2 * hd), lambda h, i: (i, h)),
        out_shape=jax.ShapeDtypeStruct((bn, Cc), jnp.float32),
        interpret=interpret,
    )(qkv, qkv, qkv, cent, rbf)

    out = pl.pallas_call(
        _proj_kernel,
        grid=grid_a,
        in_specs=[
            pl.BlockSpec((blk, Cc), lambda i: (i, 0)),
            pl.BlockSpec((Cc, Cc), lambda i: (0, 0)),
            pl.BlockSpec((1, Cc), lambda i: (0, 0)),
        ],
        out_specs=pl.BlockSpec((blk, Cc), lambda i: (i, 0)),
        out_shape=jax.ShapeDtypeStruct((bn, Cc), jnp.float32),
        interpret=interpret,
    )(of, W_proj, b_proj.reshape(1, Cc))

    return out.reshape(Bb, Nn, Cc)
